# baseline (speedup 1.0000x reference)
import functools

import jax
import jax.numpy as jnp
from jax import lax
from jax.experimental import pallas as pl
from jax.experimental.pallas import tpu as pltpu
from jax.experimental.pallas import tpu_sc as plsc

BATCH = 16384
D = 64
NREL = 100000
NC = 2
NS = 16
HALF = BATCH // NC
DPT = D // NS
CHUNK = 2048
NCHK = HALF // CHUNK
OUTW = HALF // NS

_mesh = plsc.VectorSubcoreMesh(core_axis_name="c", subcore_axis_name="s")


@functools.partial(
    pl.kernel,
    mesh=_mesh,
    out_type=jax.ShapeDtypeStruct((BATCH,), jnp.float32),
    compiler_params=pltpu.CompilerParams(needs_layout_passes=False),
    scratch_types=[
        pltpu.VMEM((NREL,), jnp.float32),
        pltpu.VMEM((HALF,), jnp.int32),
        pltpu.VMEM((2, CHUNK), jnp.float32),
        pltpu.VMEM((2, CHUNK), jnp.float32),
        pltpu.VMEM((HALF,), jnp.float32),
        pltpu.VMEM((OUTW,), jnp.float32),
        pltpu.VMEM_SHARED((NS, HALF // 2), jnp.float32),
        pltpu.SemaphoreType.DMA,
        pltpu.SemaphoreType.DMA,
        pltpu.SemaphoreType.DMA,
        pltpu.SemaphoreType.DMA,
    ],
)
def _distmult_sc(ht_hbm, rel_hbm, tt_hbm, et_hbm, out_hbm,
                 row_v, idx_v, h_v, t_v, part_v, o_v, shared_s,
                 rsem, csem0, csem1, isem):
    sc = lax.axis_index("c")
    tec = lax.axis_index("s")
    base = sc * HALF

    cp_idx = pltpu.async_copy(rel_hbm.at[pl.ds(base, HALF)], idx_v, isem)
    d0 = tec * DPT
    off = lax.rem(tec, DPT)

    def dsel(q):
        return d0 + lax.rem(q + off, DPT)

    cp_row = pltpu.async_copy(et_hbm.at[dsel(0), :], row_v, rsem)

    cp_idx.wait()

    csems = [csem0, csem1]
    NSTEP = DPT * NCHK

    def fire_ht(step):
        q, c = divmod(step, NCHK)
        s = step % 2
        cb = base + c * CHUNK
        dd = dsel(q)
        return (pltpu.async_copy(ht_hbm.at[dd, pl.ds(cb, CHUNK)],
                                 h_v.at[s], csems[s]),
                pltpu.async_copy(tt_hbm.at[dd, pl.ds(cb, CHUNK)],
                                 t_v.at[s], csems[s]))

    pend = {0: fire_ht(0)}
    for q in range(DPT):
        for c in range(NCHK):
            step = q * NCHK + c
            if step + 1 < NSTEP:
                pend[step + 1] = fire_ht(step + 1)
            cph, cpt = pend.pop(step)
            if c == 0:
                cp_row.wait()
            cph.wait()
            cpt.wait()
            s = step % 2

            @plsc.parallel_loop(0, CHUNK, step=16, unroll=4)
            def group(e, q=q, c=c, s=s):
                eoff = c * CHUNK + e
                iv = idx_v[pl.ds(eoff, 16)]
                rv = plsc.load_gather(row_v, [iv])
                v = rv * h_v[s, pl.ds(e, 16)] * t_v[s, pl.ds(e, 16)]
                if q == 0:
                    part_v[pl.ds(eoff, 16)] = v
                else:
                    part_v[pl.ds(eoff, 16)] = part_v[pl.ds(eoff, 16)] + v
        if q + 1 < DPT:
            cp_row = pltpu.async_copy(et_hbm.at[dsel(q + 1), :], row_v, rsem)

    ph_mine = tec // (NS // 2)
    for ph in range(2):
        pltpu.sync_copy(part_v.at[pl.ds(ph * (HALF // 2), HALF // 2)],
                        shared_s.at[tec])
        plsc.subcore_barrier()

        @pl.when(ph_mine == ph)
        def _read(ph=ph):
            for i in range(NS):
                pltpu.sync_copy(
                    shared_s.at[i, pl.ds(tec * OUTW - ph * (HALF // 2), OUTW)],
                    row_v.at[pl.ds(i * OUTW, OUTW)])

        plsc.subcore_barrier()

    @plsc.parallel_loop(0, OUTW, step=16)
    def osum(e):
        acc = row_v[pl.ds(e, 16)]
        for i in range(1, NS):
            acc = acc + row_v[pl.ds(i * OUTW + e, 16)]
        o_v[pl.ds(e, 16)] = acc

    pltpu.sync_copy(o_v, out_hbm.at[pl.ds(base + tec * OUTW, OUTW)])


def kernel(head, rel, tail, rel_emb):
    return _distmult_sc(head.T, rel.astype(jnp.int32), tail.T, rel_emb.T)


# speedup vs baseline: 1.0289x; 1.0289x over previous
import functools

import jax
import jax.numpy as jnp
from jax import lax
from jax.experimental import pallas as pl
from jax.experimental.pallas import tpu as pltpu
from jax.experimental.pallas import tpu_sc as plsc

BATCH = 16384
D = 64
NREL = 100000
NC = 2
NS = 16
HALF = BATCH // NC
DPT = D // NS
CHUNK = 4096
NCHK = HALF // CHUNK
OUTW = HALF // NS

_mesh = plsc.VectorSubcoreMesh(core_axis_name="c", subcore_axis_name="s")


@functools.partial(
    pl.kernel,
    mesh=_mesh,
    out_type=jax.ShapeDtypeStruct((BATCH,), jnp.float32),
    compiler_params=pltpu.CompilerParams(needs_layout_passes=False),
    scratch_types=[
        pltpu.VMEM((NREL,), jnp.float32),
        pltpu.VMEM((HALF,), jnp.int32),
        pltpu.VMEM((CHUNK,), jnp.float32),
        pltpu.VMEM((CHUNK,), jnp.float32),
        pltpu.VMEM((HALF,), jnp.float32),
        pltpu.VMEM((OUTW,), jnp.float32),
        pltpu.VMEM_SHARED((NS, HALF // 2), jnp.float32),
        pltpu.SemaphoreType.DMA,
        pltpu.SemaphoreType.DMA,
        pltpu.SemaphoreType.DMA,
    ],
)
def _distmult_sc(ht_hbm, rel_hbm, tt_hbm, et_hbm, out_hbm,
                 row_v, idx_v, h_v, t_v, part_v, o_v, shared_s,
                 rsem, csem, isem):
    sc = lax.axis_index("c")
    tec = lax.axis_index("s")
    base = sc * HALF

    cp_idx = pltpu.async_copy(rel_hbm.at[pl.ds(base, HALF)], idx_v, isem)
    d0 = tec * DPT
    off = lax.rem(tec, DPT)

    def dsel(q):
        return d0 + lax.rem(q + off, DPT)

    cp_row = pltpu.async_copy(et_hbm.at[dsel(0), :], row_v, rsem)

    cp_idx.wait()

    for q in range(DPT):
        dd = dsel(q)
        for c in range(NCHK):
            cb = base + c * CHUNK
            cp_h = pltpu.async_copy(ht_hbm.at[dd, pl.ds(cb, CHUNK)], h_v, csem)
            cp_t = pltpu.async_copy(tt_hbm.at[dd, pl.ds(cb, CHUNK)], t_v, csem)
            if c == 0:
                cp_row.wait()
            cp_h.wait()
            cp_t.wait()

            @plsc.parallel_loop(0, CHUNK, step=16, unroll=4)
            def group(e, q=q, c=c):
                eoff = c * CHUNK + e
                iv = idx_v[pl.ds(eoff, 16)]
                rv = plsc.load_gather(row_v, [iv])
                v = rv * h_v[pl.ds(e, 16)] * t_v[pl.ds(e, 16)]
                if q == 0:
                    part_v[pl.ds(eoff, 16)] = v
                else:
                    part_v[pl.ds(eoff, 16)] = part_v[pl.ds(eoff, 16)] + v
        if q + 1 < DPT:
            cp_row = pltpu.async_copy(et_hbm.at[dsel(q + 1), :], row_v, rsem)

    ph_mine = tec // (NS // 2)
    for ph in range(2):
        pltpu.sync_copy(part_v.at[pl.ds(ph * (HALF // 2), HALF // 2)],
                        shared_s.at[tec])
        plsc.subcore_barrier()

        @pl.when(ph_mine == ph)
        def _read(ph=ph):
            for i in range(NS):
                pltpu.sync_copy(
                    shared_s.at[i, pl.ds(tec * OUTW - ph * (HALF // 2), OUTW)],
                    row_v.at[pl.ds(i * OUTW, OUTW)])

        plsc.subcore_barrier()

    @plsc.parallel_loop(0, OUTW, step=16)
    def osum(e):
        acc = row_v[pl.ds(e, 16)]
        for i in range(1, NS):
            acc = acc + row_v[pl.ds(i * OUTW + e, 16)]
        o_v[pl.ds(e, 16)] = acc

    pltpu.sync_copy(o_v, out_hbm.at[pl.ds(base + tec * OUTW, OUTW)])


def kernel(head, rel, tail, rel_emb):
    return _distmult_sc(head.T, rel.astype(jnp.int32), tail.T, rel_emb.T)
